# Initial kernel scaffold; baseline (speedup 1.0000x reference)
import functools

import jax
import jax.numpy as jnp
from jax import lax
from jax.experimental import pallas as pl
from jax.experimental.pallas import tpu as pltpu
from jax.experimental.pallas import tpu_sc as plsc

N_NODES = 10000
NP = 10240
E_RAW = 320000
E_TOT = E_RAW + N_NODES
NWORK = 32
EDGE_BLK = 128
CHUNK = 10368
E_PAD = CHUNK * NWORK
NEG = -1e30
D_FEAT = 128
NHID = 64
N_CLASSES = 16


def _make_sc_edge_kernel(wf):
    mesh = plsc.VectorSubcoreMesh(core_axis_name="c", subcore_axis_name="s")
    rows_per_sub = NP // 16
    nblk = CHUNK // EDGE_BLK
    ncols = wf // 16

    @functools.partial(
        pl.kernel,
        out_type=jax.ShapeDtypeStruct((2, NP, wf), jnp.float32),
        mesh=mesh,
        scratch_types=[
            pltpu.VMEM((NP, 2), jnp.float32),
            pltpu.VMEM((EDGE_BLK,), jnp.int32),
            pltpu.VMEM((EDGE_BLK,), jnp.int32),
            pltpu.VMEM((EDGE_BLK,), jnp.float32),
            pltpu.VMEM((EDGE_BLK, wf), jnp.float32),
            pltpu.VMEM_SHARED((NP, wf), jnp.float32),
            pltpu.SemaphoreType.DMA,
        ],
    )
    def k(src_hbm, dst_hbm, sd_hbm, hext_hbm, acc_hbm,
          sd_v, sidx_v, didx_v, w_v, rows_v, acc_sh, sem):
        cid = lax.axis_index("c")
        sid = lax.axis_index("s")
        wid = cid * 16 + sid
        base = wid * CHUNK

        pltpu.sync_copy(sd_hbm, sd_v)

        @pl.loop(0, EDGE_BLK)
        def _(r):
            for c in range(ncols):
                rows_v[r, pl.ds(c * 16, 16)] = jnp.zeros((16,), jnp.float32)

        @pl.loop(0, rows_per_sub, step=EDGE_BLK)
        def _(r0):
            pltpu.sync_copy(
                rows_v, acc_sh.at[pl.ds(sid * rows_per_sub + r0, EDGE_BLK)])

        plsc.subcore_barrier()

        zeros16 = jnp.zeros((16,), jnp.int32)
        ones16 = jnp.ones((16,), jnp.int32)

        @pl.loop(0, nblk)
        def _(t):
            off = base + t * EDGE_BLK
            pltpu.sync_copy(src_hbm.at[pl.ds(off, EDGE_BLK)], sidx_v)
            pltpu.sync_copy(dst_hbm.at[pl.ds(off, EDGE_BLK)], didx_v)

            @pl.loop(0, EDGE_BLK, step=16)
            def _(j):
                si = sidx_v[pl.ds(j, 16)]
                di = didx_v[pl.ds(j, 16)]
                sval = plsc.load_gather(sd_v, [si, zeros16])
                dval = plsc.load_gather(sd_v, [di, ones16])
                e = sval + dval
                e = jnp.where(e >= 0.0, e, 0.2 * e)
                w_v[pl.ds(j, 16)] = jnp.exp(e)

            pltpu.async_copy(hext_hbm.at[sidx_v], rows_v, sem).wait()

            @pl.loop(0, EDGE_BLK)
            def _(r):
                wv = jnp.full((16,), w_v[r], jnp.float32)
                for c in range(ncols):
                    sl = pl.ds(c * 16, 16)
                    rows_v[r, sl] = rows_v[r, sl] * wv

            pltpu.sync_copy(rows_v, acc_sh.at[didx_v], add=True)

        plsc.subcore_barrier()

        pltpu.sync_copy(
            acc_sh.at[pl.ds(sid * rows_per_sub, rows_per_sub)],
            acc_hbm.at[cid].at[pl.ds(sid * rows_per_sub, rows_per_sub)])

    return k


_sc_edge_80 = _make_sc_edge_kernel(80)
_sc_edge_32 = _make_sc_edge_kernel(32)


def _valid_mask():
    return lax.broadcasted_iota(jnp.int32, (NP, 1), 0) < N_NODES


def _emit_layer_outputs(h, f_out, hext_ref, sd_ref, a):
    valid = _valid_mask()
    sd = jnp.dot(h, a, preferred_element_type=jnp.float32)
    hext_ref[:, :f_out] = jnp.where(valid, h, 0.0)
    col = lax.broadcasted_iota(jnp.int32, (NP, 16), 1)
    hext_ref[:, f_out:f_out + 16] = jnp.where(
        valid & (col == 0), 1.0, 0.0)
    sd_ref[...] = jnp.where(valid, sd, NEG)


def _dense1_body(xp_ref, w_ref, a_ref, hext_ref, sd_ref):
    h = jnp.dot(xp_ref[...], w_ref[...], preferred_element_type=jnp.float32)
    _emit_layer_outputs(h, NHID, hext_ref, sd_ref, a_ref[...])


def _make_combine_body(f_in, f_out):
    def body(acc_ref, b_ref, w_ref, a_ref, hext_ref, sd_ref):
        g = acc_ref[0] + acc_ref[1]
        num = g[:, :f_in]
        den = g[:, f_in:f_in + 1] + 1e-16
        xn = jnp.maximum(num / den + b_ref[...], 0.0)
        xn = jnp.where(_valid_mask(), xn, 0.0)
        h = jnp.dot(xn, w_ref[...], preferred_element_type=jnp.float32)
        _emit_layer_outputs(h, f_out, hext_ref, sd_ref, a_ref[...])
    return body


def _final_body(acc_ref, b_ref, out_ref):
    g = acc_ref[0] + acc_ref[1]
    o = g[:, :N_CLASSES] / (g[:, N_CLASSES:N_CLASSES + 1] + 1e-16) + b_ref[...]
    m = jnp.max(o, axis=1, keepdims=True)
    z = o - m
    out_ref[...] = z - jnp.log(jnp.sum(jnp.exp(z), axis=1, keepdims=True))


def _f32(shape):
    return jax.ShapeDtypeStruct(shape, jnp.float32)


def kernel(x, edge_index, W1, a1s, a1d, b1, W2, a2s, a2d, b2,
           W3, a3s, a3d, b3):
    ei = edge_index.astype(jnp.int32)
    loops = jnp.arange(N_NODES, dtype=jnp.int32)
    pad = jnp.full((E_PAD - E_TOT,), N_NODES, jnp.int32)
    src = jnp.concatenate([ei[0], loops, pad])
    dst = jnp.concatenate([ei[1], loops, pad])

    xp = jnp.pad(x, ((0, NP - N_NODES), (0, 0)))
    A1 = jnp.stack([a1s, a1d], axis=1)
    A2 = jnp.stack([a2s, a2d], axis=1)
    A3 = jnp.stack([a3s, a3d], axis=1)

    hext1, sd1 = pl.pallas_call(
        _dense1_body, out_shape=(_f32((NP, 80)), _f32((NP, 2))),
    )(xp, W1, A1)
    acc1 = _sc_edge_80(src, dst, sd1, hext1)

    hext2, sd2 = pl.pallas_call(
        _make_combine_body(NHID, NHID),
        out_shape=(_f32((NP, 80)), _f32((NP, 2))),
    )(acc1, b1.reshape(1, -1), W2, A2)
    acc2 = _sc_edge_80(src, dst, sd2, hext2)

    hext3, sd3 = pl.pallas_call(
        _make_combine_body(NHID, N_CLASSES),
        out_shape=(_f32((NP, 32)), _f32((NP, 2))),
    )(acc2, b2.reshape(1, -1), W3, A3)
    acc3 = _sc_edge_32(src, dst, sd3, hext3)

    out = pl.pallas_call(
        _final_body, out_shape=_f32((NP, N_CLASSES)),
    )(acc3, b3.reshape(1, -1))
    return out[:N_NODES]


# speedup vs baseline: 25.8556x; 25.8556x over previous
import functools

import jax
import jax.numpy as jnp
from jax import lax
from jax.experimental import pallas as pl
from jax.experimental.pallas import tpu as pltpu
from jax.experimental.pallas import tpu_sc as plsc

N_NODES = 10000
NP = 10240
E_RAW = 320000
E_TOT = E_RAW + N_NODES
NWORK = 32
EDGE_BLK = 128
CHUNK = 10368
E_PAD = CHUNK * NWORK
NEG = -1e30
D_FEAT = 128
NHID = 64
N_CLASSES = 16


WF = 128


def _make_sc_edge_kernel(nscale):
    mesh = plsc.VectorSubcoreMesh(core_axis_name="c", subcore_axis_name="s")
    rows_per_sub = NP // 16
    nblk = CHUNK // EDGE_BLK
    ncols = WF // 16

    @functools.partial(
        pl.kernel,
        out_type=jax.ShapeDtypeStruct((2, NP, WF), jnp.float32),
        mesh=mesh,
        scratch_types=[
            pltpu.VMEM((2 * NP,), jnp.float32),
            pltpu.VMEM((EDGE_BLK,), jnp.int32),
            pltpu.VMEM((EDGE_BLK,), jnp.int32),
            pltpu.VMEM((EDGE_BLK,), jnp.float32),
            pltpu.VMEM((EDGE_BLK, WF), jnp.float32),
            pltpu.VMEM_SHARED((NP, WF), jnp.float32),
            pltpu.SemaphoreType.DMA,
        ],
        compiler_params=pltpu.CompilerParams(needs_layout_passes=False),
    )
    def k(src_hbm, dst_hbm, sd_hbm, hext_hbm, acc_hbm,
          sd_v, sidx_v, didx_v, w_v, rows_v, acc_sh, sem):
        cid = lax.axis_index("c")
        sid = lax.axis_index("s")
        wid = cid * 16 + sid
        base = wid * CHUNK

        pltpu.sync_copy(sd_hbm, sd_v)

        @pl.loop(0, EDGE_BLK)
        def _(r):
            for c in range(ncols):
                rows_v[r, pl.ds(c * 16, 16)] = jnp.zeros((16,), jnp.float32)

        @pl.loop(0, rows_per_sub, step=EDGE_BLK)
        def _(r0):
            pltpu.sync_copy(
                rows_v, acc_sh.at[pl.ds(sid * rows_per_sub + r0, EDGE_BLK)])

        plsc.subcore_barrier()

        ones16 = jnp.ones((16,), jnp.int32)

        @pl.loop(0, nblk)
        def _(t):
            off = base + t * EDGE_BLK
            pltpu.sync_copy(src_hbm.at[pl.ds(off, EDGE_BLK)], sidx_v)
            pltpu.sync_copy(dst_hbm.at[pl.ds(off, EDGE_BLK)], didx_v)

            @pl.loop(0, EDGE_BLK, step=16)
            def _(j):
                si = sidx_v[pl.ds(j, 16)]
                di = didx_v[pl.ds(j, 16)]
                sval = plsc.load_gather(sd_v, [si + si])
                dval = plsc.load_gather(sd_v, [di + di + ones16])
                e = sval + dval
                e = jnp.where(e >= 0.0, e, 0.2 * e)
                w_v[pl.ds(j, 16)] = jnp.exp(e)

            pltpu.async_copy(hext_hbm.at[sidx_v], rows_v, sem).wait()

            @pl.loop(0, EDGE_BLK)
            def _(r):
                wv = plsc.load_gather(
                    w_v, [jnp.full((16,), r, jnp.int32)])
                for c in range(nscale):
                    sl = pl.ds(c * 16, 16)
                    rows_v[r, sl] = rows_v[r, sl] * wv

            pltpu.sync_copy(rows_v, acc_sh.at[didx_v], add=True)

        plsc.subcore_barrier()

        pltpu.sync_copy(
            acc_sh.at[pl.ds(sid * rows_per_sub, rows_per_sub)],
            acc_hbm.at[cid].at[pl.ds(sid * rows_per_sub, rows_per_sub)])

    return k


_sc_edge_wide = _make_sc_edge_kernel(5)
_sc_edge_narrow = _make_sc_edge_kernel(2)


def _valid_mask():
    return lax.broadcasted_iota(jnp.int32, (NP, 1), 0) < N_NODES


def _emit_layer_outputs(h, f_out, hext_ref, sd_ref, a):
    valid = _valid_mask()
    sd = jnp.dot(h, a, preferred_element_type=jnp.float32)
    hext_ref[:, :f_out] = jnp.where(valid, h, 0.0)
    col = lax.broadcasted_iota(jnp.int32, (NP, WF - f_out), 1)
    hext_ref[:, f_out:] = jnp.where(valid & (col == 0), 1.0, 0.0)
    sd_ref[...] = jnp.where(valid, sd, NEG)


def _dense1_body(xp_ref, w_ref, a_ref, hext_ref, sd_ref):
    h = jnp.dot(xp_ref[...], w_ref[...], preferred_element_type=jnp.float32)
    _emit_layer_outputs(h, NHID, hext_ref, sd_ref, a_ref[...])


def _make_combine_body(f_in, f_out):
    def body(acc_ref, b_ref, w_ref, a_ref, hext_ref, sd_ref):
        g = acc_ref[0] + acc_ref[1]
        num = g[:, :f_in]
        den = g[:, f_in:f_in + 1] + 1e-16
        xn = jnp.maximum(num / den + b_ref[...], 0.0)
        xn = jnp.where(_valid_mask(), xn, 0.0)
        h = jnp.dot(xn, w_ref[...], preferred_element_type=jnp.float32)
        _emit_layer_outputs(h, f_out, hext_ref, sd_ref, a_ref[...])
    return body


def _final_body(acc_ref, b_ref, out_ref):
    g = acc_ref[0] + acc_ref[1]
    o = g[:, :N_CLASSES] / (g[:, N_CLASSES:N_CLASSES + 1] + 1e-16) + b_ref[...]
    m = jnp.max(o, axis=1, keepdims=True)
    z = o - m
    out_ref[...] = z - jnp.log(jnp.sum(jnp.exp(z), axis=1, keepdims=True))


def _f32(shape):
    return jax.ShapeDtypeStruct(shape, jnp.float32)


def kernel(x, edge_index, W1, a1s, a1d, b1, W2, a2s, a2d, b2,
           W3, a3s, a3d, b3):
    ei = edge_index.astype(jnp.int32)
    loops = jnp.arange(N_NODES, dtype=jnp.int32)
    pad = jnp.full((E_PAD - E_TOT,), N_NODES, jnp.int32)
    src = jnp.concatenate([ei[0], loops, pad])
    dst = jnp.concatenate([ei[1], loops, pad])

    xp = jnp.pad(x, ((0, NP - N_NODES), (0, 0)))
    A1 = jnp.stack([a1s, a1d], axis=1)
    A2 = jnp.stack([a2s, a2d], axis=1)
    A3 = jnp.stack([a3s, a3d], axis=1)

    hext1, sd1 = pl.pallas_call(
        _dense1_body, out_shape=(_f32((NP, WF)), _f32((NP, 2))),
    )(xp, W1, A1)
    acc1 = _sc_edge_wide(src, dst, sd1.reshape(-1), hext1)

    hext2, sd2 = pl.pallas_call(
        _make_combine_body(NHID, NHID),
        out_shape=(_f32((NP, WF)), _f32((NP, 2))),
    )(acc1, b1.reshape(1, -1), W2, A2)
    acc2 = _sc_edge_wide(src, dst, sd2.reshape(-1), hext2)

    hext3, sd3 = pl.pallas_call(
        _make_combine_body(NHID, N_CLASSES),
        out_shape=(_f32((NP, WF)), _f32((NP, 2))),
    )(acc2, b2.reshape(1, -1), W3, A3)
    acc3 = _sc_edge_narrow(src, dst, sd3.reshape(-1), hext3)

    out = pl.pallas_call(
        _final_body, out_shape=_f32((NP, N_CLASSES)),
    )(acc3, b3.reshape(1, -1))
    return out[:N_NODES]
